# baseline (speedup 1.0000x reference)
import functools

import jax
import jax.numpy as jnp
from jax.experimental import pallas as pl


def _split3(x):
    b0 = x.astype(jnp.bfloat16)
    r = x - b0.astype(jnp.float32)
    b1 = r.astype(jnp.bfloat16)
    r2 = r - b1.astype(jnp.float32)
    b2 = r2.astype(jnp.bfloat16)
    return b0, b1, b2


def _augment(coords, sq, lhs):
    b0, b1, b2 = _split3(coords)
    cols = []
    for d in range(3):
        parts = (b0, b0, b0, b1, b1, b2) if lhs else (b0, b1, b2, b0, b1, b0)
        cols.extend(p[..., d:d + 1] for p in parts)
    s0, s1, s2 = _split3(sq)
    one = jnp.ones_like(s0)
    cols.extend([s0, s1, s2, one, one, one] if lhs else [one, one, one, s0, s1, s2])
    return jnp.concatenate(cols, axis=-1)


def _chamfer_body(a1_ref, a2_ref, d1_ref, d2_ref):
    j = pl.program_id(1)

    pd = jax.lax.dot_general(
        a1_ref[0], a2_ref[0],
        dimension_numbers=(((1,), (1,)), ((), ())),
        preferred_element_type=jnp.float32,
    )

    rowmin = jnp.min(pd, axis=1)
    d2_ref[0, 0] = jnp.min(pd, axis=0)

    @pl.when(j == 0)
    def _():
        d1_ref[0, 0] = rowmin

    @pl.when(j != 0)
    def _():
        d1_ref[0, 0] = jnp.minimum(d1_ref[0, 0], rowmin)


@functools.partial(jax.jit, static_argnames=("interpret",))
def _chamfer(xyz1, xyz2, interpret=False):
    B, N, _ = xyz1.shape
    M = xyz2.shape[1]
    TM = 1024

    sq1 = jnp.sum(xyz1 * xyz1, axis=-1, keepdims=True)
    sq2 = jnp.sum(xyz2 * xyz2, axis=-1, keepdims=True)
    a1 = _augment(xyz1, sq1, lhs=True)
    a2 = _augment(-2.0 * xyz2, sq2, lhs=False)

    grid = (B, M // TM)
    d1, d2 = pl.pallas_call(
        _chamfer_body,
        grid=grid,
        in_specs=[
            pl.BlockSpec((1, N, 24), lambda b, j: (b, 0, 0)),
            pl.BlockSpec((1, TM, 24), lambda b, j: (b, j, 0)),
        ],
        out_specs=[
            pl.BlockSpec((1, 1, N), lambda b, j: (b, 0, 0)),
            pl.BlockSpec((1, 1, TM), lambda b, j: (b, 0, j)),
        ],
        out_shape=[
            jax.ShapeDtypeStruct((B, 1, N), jnp.float32),
            jax.ShapeDtypeStruct((B, 1, M), jnp.float32),
        ],
        interpret=interpret,
    )(a1, a2)
    return d1, d2


def kernel(xyz1, xyz2):
    if xyz1.ndim == 2:
        xyz1 = xyz1[None]
    if xyz2.ndim == 2:
        xyz2 = xyz2[None]
    d1, d2 = _chamfer(xyz1, xyz2)
    return (d1[:, 0, :], d2[:, 0, :])


# speedup vs baseline: 1.0567x; 1.0567x over previous
import functools

import jax
import jax.numpy as jnp
from jax.experimental import pallas as pl
from jax.experimental.pallas import tpu as pltpu


def _split3(x):
    b0 = x.astype(jnp.bfloat16)
    r = x - b0.astype(jnp.float32)
    b1 = r.astype(jnp.bfloat16)
    r2 = r - b1.astype(jnp.float32)
    b2 = r2.astype(jnp.bfloat16)
    return b0, b1, b2


def _augment(coords, sq, lhs):
    b0, b1, b2 = _split3(coords)
    cols = []
    for d in range(3):
        parts = (b0, b0, b0, b1, b1, b2) if lhs else (b0, b1, b2, b0, b1, b0)
        cols.extend(p[..., d:d + 1] for p in parts)
    s0, s1, s2 = _split3(sq)
    one = jnp.ones_like(s0)
    cols.extend([s0, s1, s2, one, one, one] if lhs else [one, one, one, s0, s1, s2])
    return jnp.concatenate(cols, axis=-1)


def _chamfer_body(a1_ref, a2_ref, d1_ref, d2_ref, racc_ref):
    j = pl.program_id(1)
    nj = pl.num_programs(1)

    pd = jax.lax.dot_general(
        a1_ref[0], a2_ref[0],
        dimension_numbers=(((1,), (1,)), ((), ())),
        preferred_element_type=jnp.float32,
    )

    tm = pd.shape[1]
    rp = pd[:, 0:128]
    for k in range(1, tm // 128):
        rp = jnp.minimum(rp, pd[:, k * 128:(k + 1) * 128])

    d2_ref[0, 0] = jnp.min(pd, axis=0)

    @pl.when(j == 0)
    def _():
        racc_ref[...] = rp

    @pl.when(j != 0)
    def _():
        racc_ref[...] = jnp.minimum(racc_ref[...], rp)

    @pl.when(j == nj - 1)
    def _():
        d1_ref[0, 0] = jnp.min(racc_ref[...], axis=1)


@functools.partial(jax.jit, static_argnames=("interpret",))
def _chamfer(xyz1, xyz2, interpret=False):
    B, N, _ = xyz1.shape
    M = xyz2.shape[1]
    TM = 1024

    sq1 = jnp.sum(xyz1 * xyz1, axis=-1, keepdims=True)
    sq2 = jnp.sum(xyz2 * xyz2, axis=-1, keepdims=True)
    a1 = _augment(xyz1, sq1, lhs=True)
    a2 = _augment(-2.0 * xyz2, sq2, lhs=False)

    grid = (B, M // TM)
    d1, d2 = pl.pallas_call(
        _chamfer_body,
        grid=grid,
        in_specs=[
            pl.BlockSpec((1, N, 24), lambda b, j: (b, 0, 0)),
            pl.BlockSpec((1, TM, 24), lambda b, j: (b, j, 0)),
        ],
        out_specs=[
            pl.BlockSpec((1, 1, N), lambda b, j: (b, 0, 0)),
            pl.BlockSpec((1, 1, TM), lambda b, j: (b, 0, j)),
        ],
        out_shape=[
            jax.ShapeDtypeStruct((B, 1, N), jnp.float32),
            jax.ShapeDtypeStruct((B, 1, M), jnp.float32),
        ],
        scratch_shapes=[pltpu.VMEM((N, 128), jnp.float32)],
        interpret=interpret,
    )(a1, a2)
    return d1, d2


def kernel(xyz1, xyz2):
    if xyz1.ndim == 2:
        xyz1 = xyz1[None]
    if xyz2.ndim == 2:
        xyz2 = xyz2[None]
    d1, d2 = _chamfer(xyz1, xyz2)
    return (d1[:, 0, :], d2[:, 0, :])


# speedup vs baseline: 5.0993x; 4.8259x over previous
import functools

import jax
import jax.numpy as jnp
from jax.experimental import pallas as pl
from jax.experimental.pallas import tpu as pltpu


def _split3(x):
    b0 = x.astype(jnp.bfloat16)
    r = x - b0.astype(jnp.float32)
    b1 = r.astype(jnp.bfloat16)
    r2 = r - b1.astype(jnp.float32)
    b2 = r2.astype(jnp.bfloat16)
    return b0, b1, b2


def _augment(coords, sq, lhs):
    b0, b1, b2 = _split3(coords)
    cols = []
    for d in range(3):
        parts = (b0, b0, b0, b1, b1, b2) if lhs else (b0, b1, b2, b0, b1, b0)
        cols.extend(p[..., d:d + 1] for p in parts)
    s0, s1, s2 = _split3(sq)
    one = jnp.ones_like(s0)
    cols.extend([s0, s1, s2, one, one, one] if lhs else [one, one, one, s0, s1, s2])
    return jnp.concatenate(cols, axis=-1)


def _chamfer_body(a1_ref, a2_ref, d1_ref, d2_ref, racc_ref):
    j = pl.program_id(1)
    nj = pl.num_programs(1)

    pd = jax.lax.dot_general(
        a1_ref[0], a2_ref[0],
        dimension_numbers=(((1,), (0,)), ((), ())),
        preferred_element_type=jnp.float32,
    )

    tm = pd.shape[1]
    rp = pd[:, 0:128]
    for k in range(1, tm // 128):
        rp = jnp.minimum(rp, pd[:, k * 128:(k + 1) * 128])

    d2_ref[0, 0] = jnp.min(pd, axis=0)

    @pl.when(j == 0)
    def _():
        racc_ref[...] = rp

    @pl.when(j != 0)
    def _():
        racc_ref[...] = jnp.minimum(racc_ref[...], rp)

    @pl.when(j == nj - 1)
    def _():
        d1_ref[0, 0] = jnp.min(racc_ref[...], axis=1)


@functools.partial(jax.jit, static_argnames=("interpret",))
def _chamfer(xyz1, xyz2, interpret=False):
    B, N, _ = xyz1.shape
    M = xyz2.shape[1]
    TM = 1024

    sq1 = jnp.sum(xyz1 * xyz1, axis=-1, keepdims=True)
    sq2 = jnp.sum(xyz2 * xyz2, axis=-1, keepdims=True)
    a1 = _augment(xyz1, sq1, lhs=True)
    a2 = _augment(-2.0 * xyz2, sq2, lhs=False).transpose(0, 2, 1)

    grid = (B, M // TM)
    d1, d2 = pl.pallas_call(
        _chamfer_body,
        grid=grid,
        in_specs=[
            pl.BlockSpec((1, N, 24), lambda b, j: (b, 0, 0)),
            pl.BlockSpec((1, 24, TM), lambda b, j: (b, 0, j)),
        ],
        out_specs=[
            pl.BlockSpec((1, 1, N), lambda b, j: (b, 0, 0)),
            pl.BlockSpec((1, 1, TM), lambda b, j: (b, 0, j)),
        ],
        out_shape=[
            jax.ShapeDtypeStruct((B, 1, N), jnp.float32),
            jax.ShapeDtypeStruct((B, 1, M), jnp.float32),
        ],
        scratch_shapes=[pltpu.VMEM((N, 128), jnp.float32)],
        interpret=interpret,
    )(a1, a2)
    return d1, d2


def kernel(xyz1, xyz2):
    if xyz1.ndim == 2:
        xyz1 = xyz1[None]
    if xyz2.ndim == 2:
        xyz2 = xyz2[None]
    d1, d2 = _chamfer(xyz1, xyz2)
    return (d1[:, 0, :], d2[:, 0, :])
